# baseline (speedup 1.0000x reference)
import jax
import jax.numpy as jnp
from jax import lax
from jax.experimental import pallas as pl
from jax.experimental.pallas import tpu as pltpu
from jax.experimental.pallas import tpu_sc as plsc

VOCAB = 1000000
EMBED = 64
BATCH = 4096
HIST = 200

NUM_CORES = 2
NUM_SUBCORES = 16
NUM_WORKERS = NUM_CORES * NUM_SUBCORES
ROWS_PER_WORKER = BATCH // NUM_WORKERS
CHUNK_ROWS = 2
CHUNK_IDS = CHUNK_ROWS * HIST
NUM_CHUNKS = ROWS_PER_WORKER // CHUNK_ROWS
LANES = 16
VPR = EMBED // LANES
PACK = 2 * EMBED
HALF_V = VOCAB // 2
OUT_W = ROWS_PER_WORKER * EMBED

_splits = []
_off = 0
while _off < CHUNK_IDS:
    _sz = min(128, CHUNK_IDS - _off)
    _splits.append((_off, _sz))
    _off += _sz
GATHER_SPLITS = tuple(_splits)


def _pool_kernel(ids_hbm, packed_hbm, out_hbm,
                 idx0, idx1, pidx0, pidx1, rows0, rows1, out_loc,
                 sem0, sem1):
    wid = lax.axis_index("s") * NUM_CORES + lax.axis_index("c")
    ids_base = wid * ROWS_PER_WORKER * HIST
    idxs, pidxs, rows, sems = (
        (idx0, idx1), (pidx0, pidx1), (rows0, rows1), (sem0, sem1))

    col = [lax.iota(jnp.int32, LANES) + v * LANES for v in range(VPR)]

    def stage(c, buf):
        pltpu.sync_copy(
            ids_hbm.at[pl.ds(ids_base + c * CHUNK_IDS, CHUNK_IDS)],
            idxs[buf],
        )

        @pl.loop(0, CHUNK_IDS // LANES)
        def _split(k):
            iv = col[0] + k * LANES
            v = plsc.load_gather(idxs[buf], [iv])
            plsc.store_scatter(pidxs[buf], [iv],
                               jnp.where(v >= HALF_V, v - HALF_V, v))

        for off, sz in GATHER_SPLITS:
            pltpu.async_copy(
                packed_hbm.at[pidxs[buf].at[pl.ds(off, sz)]],
                rows[buf].at[pl.ds(off, sz)],
                sems[buf],
            )

    def drain(buf):
        pltpu.make_async_copy(
            packed_hbm.at[pl.ds(0, CHUNK_IDS)], rows[buf], sems[buf]
        ).wait()

    def pool(c, buf):
        for r in range(CHUNK_ROWS):
            zeros = tuple(jnp.zeros((LANES,), jnp.float32)
                          for _ in range(VPR))

            @pl.loop(0, HIST // 8, init_carry=zeros)
            def accs(k, acc):
                idv = plsc.load_gather(idxs[buf], [col[0] + (r * HIST + k * 8)])
                pfi = jnp.where(idv >= HALF_V,
                                jnp.full((LANES,), EMBED, jnp.int32),
                                jnp.full((LANES,), 0, jnp.int32))
                for l in range(8):
                    j = r * HIST + k * 8 + l
                    jvec = jnp.full((LANES,), 0, jnp.int32) + j
                    ps = jnp.take_along_axis(
                        pfi, jnp.full((LANES,), l, jnp.int32), axis=0)
                    acc = tuple(
                        acc[v] + plsc.load_gather(rows[buf],
                                                  [jvec, col[v] + ps])
                        for v in range(VPR)
                    )
                return acc

            obase = (c * CHUNK_ROWS + r) * EMBED
            for v in range(VPR):
                plsc.store_scatter(out_loc, [col[0] + (obase + v * LANES)],
                                   accs[v] * (1.0 / HIST))

    stage(0, 0)

    @pl.loop(0, NUM_CHUNKS, step=2)
    def _main(cc):
        for b in range(2):
            c = cc + b
            if b == 0:
                stage(cc + 1, 1)
            else:
                @pl.when(cc < NUM_CHUNKS - 2)
                def _():
                    stage(cc + 2, 0)
            drain(b)
            pool(c, b)

    pltpu.sync_copy(out_loc, out_hbm.at[pl.ds(wid * OUT_W, OUT_W)])


@jax.jit
def _pooled_means(ids_flat, packed_table):
    mesh = plsc.VectorSubcoreMesh(core_axis_name="c", subcore_axis_name="s")
    return pl.kernel(
        _pool_kernel,
        out_type=jax.ShapeDtypeStruct((BATCH * EMBED,), jnp.float32),
        mesh=mesh,
        compiler_params=pltpu.CompilerParams(needs_layout_passes=False),
        scratch_types=[
            pltpu.VMEM((CHUNK_IDS,), jnp.int32),
            pltpu.VMEM((CHUNK_IDS,), jnp.int32),
            pltpu.VMEM((CHUNK_IDS,), jnp.int32),
            pltpu.VMEM((CHUNK_IDS,), jnp.int32),
            pltpu.VMEM((CHUNK_IDS, PACK), jnp.float32),
            pltpu.VMEM((CHUNK_IDS, PACK), jnp.float32),
            pltpu.VMEM((OUT_W,), jnp.float32),
            pltpu.SemaphoreType.DMA,
            pltpu.SemaphoreType.DMA,
        ],
    )(ids_flat, packed_table)


REPACK_BLOCK = 4000
N_RB = HALF_V // REPACK_BLOCK


def _repack_kernel(x_ref, o_ref):
    g = pl.program_id(1)

    @pl.when(g == 0)
    def _():
        o_ref[:, 0:EMBED] = x_ref[...]

    @pl.when(g == 1)
    def _():
        o_ref[:, EMBED:PACK] = x_ref[...]


@jax.jit
def _repack(emb_table):
    return pl.pallas_call(
        _repack_kernel,
        out_shape=jax.ShapeDtypeStruct((HALF_V, PACK), jnp.float32),
        grid=(N_RB, 2),
        in_specs=[
            pl.BlockSpec((REPACK_BLOCK, EMBED), lambda i, g: (i + g * N_RB, 0)),
        ],
        out_specs=pl.BlockSpec((REPACK_BLOCK, PACK), lambda i, g: (i, 0)),
    )(emb_table)


def _proj_kernel(x_ref, w_ref, b_ref, o_ref):
    o_ref[...] = (
        lax.dot_general(
            x_ref[...], w_ref[...],
            (((1,), (1,)), ((), ())),
            preferred_element_type=jnp.float32,
        )
        + b_ref[...]
    )


@jax.jit
def _project(pooled, W, b2d):
    return pl.pallas_call(
        _proj_kernel,
        out_shape=jax.ShapeDtypeStruct((BATCH, EMBED), jnp.float32),
    )(pooled, W, b2d)


def kernel(input_ids, emb_table, W, b):
    ids_flat = input_ids.reshape(-1).astype(jnp.int32)
    packed_table = _repack(emb_table)
    pooled = _pooled_means(ids_flat, packed_table).reshape(BATCH, EMBED)
    return _project(pooled, W, b.reshape(1, EMBED))


# speedup vs baseline: 1.1550x; 1.1550x over previous
import jax
import jax.numpy as jnp
from jax import lax
from jax.experimental import pallas as pl
from jax.experimental.pallas import tpu as pltpu
from jax.experimental.pallas import tpu_sc as plsc

VOCAB = 1000000
EMBED = 64
BATCH = 4096
HIST = 200

NUM_CORES = 2
NUM_SUBCORES = 16
NUM_WORKERS = NUM_CORES * NUM_SUBCORES
ROWS_PER_WORKER = BATCH // NUM_WORKERS
CHUNK_ROWS = 2
CHUNK_IDS = CHUNK_ROWS * HIST
NUM_CHUNKS = ROWS_PER_WORKER // CHUNK_ROWS
LANES = 16
VPR = EMBED // LANES
PACK = 2 * EMBED
HALF_V = VOCAB // 2
OUT_W = ROWS_PER_WORKER * EMBED

_splits = []
_off = 0
while _off < CHUNK_IDS:
    _sz = min(128, CHUNK_IDS - _off)
    _splits.append((_off, _sz))
    _off += _sz
GATHER_SPLITS = tuple(_splits)


def _pool_kernel(ids_hbm, packed_hbm, out_hbm,
                 idx0, idx1, pidx0, pidx1, rows0, rows1, out_loc,
                 sem0, sem1):
    wid = lax.axis_index("s") * NUM_CORES + lax.axis_index("c")
    ids_base = wid * ROWS_PER_WORKER * HIST
    idxs, pidxs, rows, sems = (
        (idx0, idx1), (pidx0, pidx1), (rows0, rows1), (sem0, sem1))

    col = [lax.iota(jnp.int32, LANES) + v * LANES for v in range(VPR)]

    def stage(c, buf):
        pltpu.sync_copy(
            ids_hbm.at[pl.ds(ids_base + c * CHUNK_IDS, CHUNK_IDS)],
            idxs[buf],
        )

        @pl.loop(0, CHUNK_IDS // LANES)
        def _split(k):
            iv = col[0] + k * LANES
            v = plsc.load_gather(idxs[buf], [iv])
            plsc.store_scatter(pidxs[buf], [iv],
                               lax.shift_right_logical(v, 1))

        for off, sz in GATHER_SPLITS:
            pltpu.async_copy(
                packed_hbm.at[pidxs[buf].at[pl.ds(off, sz)]],
                rows[buf].at[pl.ds(off, sz)],
                sems[buf],
            )

    def drain(buf):
        pltpu.make_async_copy(
            packed_hbm.at[pl.ds(0, CHUNK_IDS)], rows[buf], sems[buf]
        ).wait()

    def pool(c, buf):
        for r in range(CHUNK_ROWS):
            zeros = tuple(jnp.zeros((LANES,), jnp.float32)
                          for _ in range(VPR))

            @pl.loop(0, HIST // 8, init_carry=zeros)
            def accs(k, acc):
                idv = plsc.load_gather(idxs[buf], [col[0] + (r * HIST + k * 8)])
                pfi = lax.shift_left((idv & 1), 6)
                for l in range(8):
                    j = r * HIST + k * 8 + l
                    jvec = jnp.full((LANES,), 0, jnp.int32) + j
                    ps = jnp.take_along_axis(
                        pfi, jnp.full((LANES,), l, jnp.int32), axis=0)
                    acc = tuple(
                        acc[v] + plsc.load_gather(rows[buf],
                                                  [jvec, col[v] + ps])
                        for v in range(VPR)
                    )
                return acc

            obase = (c * CHUNK_ROWS + r) * EMBED
            for v in range(VPR):
                plsc.store_scatter(out_loc, [col[0] + (obase + v * LANES)],
                                   accs[v] * (1.0 / HIST))

    stage(0, 0)

    @pl.loop(0, NUM_CHUNKS, step=2)
    def _main(cc):
        for b in range(2):
            c = cc + b
            if b == 0:
                stage(cc + 1, 1)
            else:
                @pl.when(cc < NUM_CHUNKS - 2)
                def _():
                    stage(cc + 2, 0)
            drain(b)
            pool(c, b)

    pltpu.sync_copy(out_loc, out_hbm.at[pl.ds(wid * OUT_W, OUT_W)])


@jax.jit
def _pooled_means(ids_flat, packed_table):
    mesh = plsc.VectorSubcoreMesh(core_axis_name="c", subcore_axis_name="s")
    return pl.kernel(
        _pool_kernel,
        out_type=jax.ShapeDtypeStruct((BATCH * EMBED,), jnp.float32),
        mesh=mesh,
        compiler_params=pltpu.CompilerParams(needs_layout_passes=False),
        scratch_types=[
            pltpu.VMEM((CHUNK_IDS,), jnp.int32),
            pltpu.VMEM((CHUNK_IDS,), jnp.int32),
            pltpu.VMEM((CHUNK_IDS,), jnp.int32),
            pltpu.VMEM((CHUNK_IDS,), jnp.int32),
            pltpu.VMEM((CHUNK_IDS, PACK), jnp.float32),
            pltpu.VMEM((CHUNK_IDS, PACK), jnp.float32),
            pltpu.VMEM((OUT_W,), jnp.float32),
            pltpu.SemaphoreType.DMA,
            pltpu.SemaphoreType.DMA,
        ],
    )(ids_flat, packed_table)


def _proj_kernel(x_ref, w_ref, b_ref, o_ref):
    o_ref[...] = (
        lax.dot_general(
            x_ref[...], w_ref[...],
            (((1,), (1,)), ((), ())),
            preferred_element_type=jnp.float32,
        )
        + b_ref[...]
    )


@jax.jit
def _project(pooled, W, b2d):
    return pl.pallas_call(
        _proj_kernel,
        out_shape=jax.ShapeDtypeStruct((BATCH, EMBED), jnp.float32),
    )(pooled, W, b2d)


def kernel(input_ids, emb_table, W, b):
    ids_flat = input_ids.reshape(-1).astype(jnp.int32)
    packed_table = emb_table.reshape(VOCAB // 2, PACK)
    pooled = _pooled_means(ids_flat, packed_table).reshape(BATCH, EMBED)
    return _project(pooled, W, b.reshape(1, EMBED))
